# baseline (speedup 1.0000x reference)
import jax
import jax.numpy as jnp
import numpy as np
from jax.experimental import pallas as pl
from jax.experimental.pallas import tpu as pltpu

_V = 204
_PAD = _V - 1
_HALF = (_V - 4) / 2.0
_SPLIT = 101

_B = 64
_N = 2048
_T3 = 3 * _N
_TRI = 1024
_TSLABS = _TRI // 128
_GRID_T = _N // _TRI


def _tok_value(xt, col):
    m = jnp.max(xt, axis=1, keepdims=True)
    tok = jnp.min(jnp.where(xt == m, col, _V), axis=1, keepdims=True)
    return tok.astype(jnp.float32).reshape(_TSLABS, 128) / _HALF - 1.0


def _smooth_l1(p, tgt):
    d = p - tgt
    ad = jnp.abs(d)
    return jnp.where(ad < 1.0, 0.5 * d * d, ad - 0.5)


def _loss_kernel(xa_ref, xs_ref, xr_ref, ga_ref, gs_ref, gr_ref, w_ref,
                 out_ref):
    t = pl.program_id(1)
    xa = xa_ref[0, 0]
    xs = xs_ref[0, 0]
    xr = xr_ref[0, 0]

    col = jax.lax.broadcasted_iota(jnp.int32, (_TRI, _V), 1)
    sl_a = _smooth_l1(jnp.abs(_tok_value(xa, col)), ga_ref[0, 0])
    sl_s = _smooth_l1(_tok_value(xs, col), gs_ref[0, 0])

    mr = jnp.max(xr, axis=1, keepdims=True)
    e = jnp.exp(xr - mr)
    sums = jax.lax.dot(e, w_ref[...],
                       preferred_element_type=jnp.float32)
    s_no = sums[:, 0:1].reshape(_TSLABS, 128)
    s_tot = sums[:, 1:2].reshape(_TSLABS, 128)
    inv = 1.0 / s_tot
    p_no = s_no * inv
    p_yes = (s_tot - s_no) * inv
    lse = jnp.logaddexp(p_no, p_yes)
    gr = gr_ref[0, 0]
    chosen = jnp.where(gr == 0, p_no, p_yes)
    nll = lse - chosen
    valid = (gr != _PAD).astype(jnp.float32)

    upd = jnp.stack([sl_a, sl_s, valid * nll, valid], axis=0)

    @pl.when(t == 0)
    def _():
        out_ref[0] = upd

    @pl.when(t != 0)
    def _():
        out_ref[0] += upd


def kernel(pred, gt_acc, gt_steer, gt_reverse):
    pc = pred[:, :_T3, :].reshape(_B, _N, 3, _V).transpose(0, 2, 1, 3)

    ga = gt_acc.reshape(_B, _GRID_T, _TSLABS, 128)
    gs = gt_steer.reshape(_B, _GRID_T, _TSLABS, 128)
    gr = gt_reverse.astype(jnp.int32).reshape(_B, _GRID_T, _TSLABS, 128)

    w = np.zeros((_V, 128), dtype=np.float32)
    w[:_SPLIT, 0] = 1.0
    w[:, 1] = 1.0
    w = jnp.asarray(w)

    def xspec(k):
        return pl.BlockSpec((1, 1, _TRI, _V), lambda b, t, k=k: (b, k, t, 0))

    gspec = pl.BlockSpec((1, 1, _TSLABS, 128), lambda b, t: (b, t, 0, 0))
    out = pl.pallas_call(
        _loss_kernel,
        grid=(_B, _GRID_T),
        in_specs=[
            xspec(0), xspec(1), xspec(2),
            gspec, gspec, gspec,
            pl.BlockSpec((_V, 128), lambda b, t: (0, 0)),
        ],
        out_specs=pl.BlockSpec((1, 4, _TSLABS, 128), lambda b, t: (b, 0, 0, 0)),
        out_shape=jax.ShapeDtypeStruct((_B, 4, _TSLABS, 128), jnp.float32),
        compiler_params=pltpu.CompilerParams(
            dimension_semantics=("parallel", "arbitrary")),
    )(pc, pc, pc, ga, gs, gr, w)

    sums = jnp.sum(out, axis=(0, 2, 3))
    acc_steer_val_loss = (sums[0] + sums[1]) / float(_B * _N)
    reverse_val_loss = sums[2] / jnp.maximum(sums[3], 1.0)
    return acc_steer_val_loss, reverse_val_loss


# speedup vs baseline: 1.2975x; 1.2975x over previous
import jax
import jax.numpy as jnp
import numpy as np
from jax.experimental import pallas as pl
from jax.experimental.pallas import tpu as pltpu

_V = 204
_PAD = _V - 1
_HALF = (_V - 4) / 2.0
_SPLIT = 101

_B = 64
_N = 2048
_T3 = 3 * _N
_ROWS = 3072
_SLABS = _ROWS // 128
_GRID_T = _T3 // _ROWS


def _loss_kernel(x_ref, tgt_ref, w_ref, out_ref):
    t = pl.program_id(1)
    x = x_ref[0]
    tgt = tgt_ref[0]

    col = jax.lax.broadcasted_iota(jnp.int32, (_ROWS, _V), 1)
    m = jnp.max(x, axis=1, keepdims=True)
    tok = jnp.min(jnp.where(x == m, col, _V), axis=1, keepdims=True)
    e = jnp.exp(x - m)
    sums = jax.lax.dot(e, w_ref[...],
                       preferred_element_type=jnp.float32)

    tokf = tok.astype(jnp.float32).reshape(_SLABS, 128) / _HALF - 1.0
    s_no = sums[:, 0:1].reshape(_SLABS, 128)
    s_tot = sums[:, 1:2].reshape(_SLABS, 128)
    q = s_no / s_tot

    r = (jax.lax.broadcasted_iota(jnp.int32, (_SLABS, 128), 0) * 128
         + jax.lax.broadcasted_iota(jnp.int32, (_SLABS, 128), 1))
    rm = r % 3
    pv = jnp.where(rm == 0, jnp.abs(tokf), tokf)
    d = pv - tgt
    ad = jnp.abs(d)
    sl = jnp.where(ad < 1.0, 0.5 * d * d, ad - 0.5)
    p_yes = 1.0 - q
    lse = jnp.logaddexp(q, p_yes)
    chosen = jnp.where(tgt == 0.0, q, p_yes)
    nll = lse - chosen
    valid = jnp.logical_and(rm == 2, tgt != float(_PAD)).astype(jnp.float32)

    zero = jnp.zeros_like(sl)
    upd = jnp.stack(
        [jnp.where(rm == 0, sl, zero),
         jnp.where(rm == 1, sl, zero),
         valid * nll,
         valid],
        axis=0)

    @pl.when(t == 0)
    def _():
        out_ref[0] = upd

    @pl.when(t != 0)
    def _():
        out_ref[0] += upd


def kernel(pred, gt_acc, gt_steer, gt_reverse):
    tgt = jnp.stack(
        [gt_acc, gt_steer, gt_reverse.astype(jnp.float32)], axis=-1
    ).reshape(_B, _GRID_T * _SLABS, 128)

    w = np.zeros((_V, 128), dtype=np.float32)
    w[:_SPLIT, 0] = 1.0
    w[:, 1] = 1.0
    w = jnp.asarray(w)

    out = pl.pallas_call(
        _loss_kernel,
        grid=(_B, _GRID_T),
        in_specs=[
            pl.BlockSpec((1, _ROWS, _V), lambda b, t: (b, t, 0)),
            pl.BlockSpec((1, _SLABS, 128), lambda b, t: (b, t, 0)),
            pl.BlockSpec((_V, 128), lambda b, t: (0, 0)),
        ],
        out_specs=pl.BlockSpec((1, 4, _SLABS, 128), lambda b, t: (b, 0, 0, 0)),
        out_shape=jax.ShapeDtypeStruct((_B, 4, _SLABS, 128), jnp.float32),
        compiler_params=pltpu.CompilerParams(
            dimension_semantics=("parallel", "arbitrary")),
    )(pred, tgt, w)

    sums = jnp.sum(out, axis=(0, 2, 3))
    acc_steer_val_loss = (sums[0] + sums[1]) / float(_B * _N)
    reverse_val_loss = sums[2] / jnp.maximum(sums[3], 1.0)
    return acc_steer_val_loss, reverse_val_loss


# speedup vs baseline: 1.3498x; 1.0403x over previous
import jax
import jax.numpy as jnp
import numpy as np
from jax.experimental import pallas as pl
from jax.experimental.pallas import tpu as pltpu

_V = 204
_PAD = _V - 1
_HALF = (_V - 4) / 2.0
_SPLIT = 101

_B = 64
_N = 2048
_T3 = 3 * _N
_ROWS = 3072
_SLABS = _ROWS // 128
_GRID_T = _T3 // _ROWS


def _loss_kernel(x_ref, tgt_ref, w_ref, out_ref):
    t = pl.program_id(1)
    x = x_ref[0]
    tgt = tgt_ref[0]

    col = jax.lax.broadcasted_iota(jnp.int32, (_ROWS, _V), 1)
    m = jnp.max(x, axis=1, keepdims=True)
    tok = jnp.min(jnp.where(x == m, col, _V), axis=1, keepdims=True)
    e = jnp.exp(x - m)
    sums = jax.lax.dot(e, w_ref[...],
                       preferred_element_type=jnp.float32)

    tokf = (tok.astype(jnp.bfloat16).reshape(_SLABS, 128).astype(jnp.float32)
            / _HALF - 1.0)
    q_rows = sums[:, 0:1] / sums[:, 1:2]
    q = (q_rows.astype(jnp.bfloat16).reshape(_SLABS, 128)
         .astype(jnp.float32))

    r = (jax.lax.broadcasted_iota(jnp.int32, (_SLABS, 128), 0) * 128
         + jax.lax.broadcasted_iota(jnp.int32, (_SLABS, 128), 1))
    rm = r % 3
    pv = jnp.where(rm == 0, jnp.abs(tokf), tokf)
    d = pv - tgt
    ad = jnp.abs(d)
    sl = jnp.where(ad < 1.0, 0.5 * d * d, ad - 0.5)
    p_yes = 1.0 - q
    lse = jnp.logaddexp(q, p_yes)
    chosen = jnp.where(tgt == 0.0, q, p_yes)
    nll = lse - chosen
    valid = jnp.logical_and(rm == 2, tgt != float(_PAD)).astype(jnp.float32)

    zero = jnp.zeros_like(sl)
    upd = jnp.stack(
        [jnp.where(rm == 0, sl, zero),
         jnp.where(rm == 1, sl, zero),
         valid * nll,
         valid],
        axis=0)

    @pl.when(t == 0)
    def _():
        out_ref[0] = upd

    @pl.when(t != 0)
    def _():
        out_ref[0] += upd


def kernel(pred, gt_acc, gt_steer, gt_reverse):
    tgt = jnp.stack(
        [gt_acc, gt_steer, gt_reverse.astype(jnp.float32)], axis=-1
    ).reshape(_B, _GRID_T * _SLABS, 128)

    w = np.zeros((_V, 128), dtype=np.float32)
    w[:_SPLIT, 0] = 1.0
    w[:, 1] = 1.0
    w = jnp.asarray(w)

    out = pl.pallas_call(
        _loss_kernel,
        grid=(_B, _GRID_T),
        in_specs=[
            pl.BlockSpec((1, _ROWS, _V), lambda b, t: (b, t, 0)),
            pl.BlockSpec((1, _SLABS, 128), lambda b, t: (b, t, 0)),
            pl.BlockSpec((_V, 128), lambda b, t: (0, 0)),
        ],
        out_specs=pl.BlockSpec((1, 4, _SLABS, 128), lambda b, t: (b, 0, 0, 0)),
        out_shape=jax.ShapeDtypeStruct((_B, 4, _SLABS, 128), jnp.float32),
        compiler_params=pltpu.CompilerParams(
            dimension_semantics=("parallel", "arbitrary")),
    )(pred, tgt, w)

    sums = jnp.sum(out, axis=(0, 2, 3))
    acc_steer_val_loss = (sums[0] + sums[1]) / float(_B * _N)
    reverse_val_loss = sums[2] / jnp.maximum(sums[3], 1.0)
    return acc_steer_val_loss, reverse_val_loss
